# Initial kernel scaffold; baseline (speedup 1.0000x reference)
import jax
import jax.numpy as jnp
from jax.experimental import pallas as pl


def kernel(input, table):
    raise NotImplementedError("write your pallas kernel here")


# speedup vs baseline: 1.8378x; 1.8378x over previous
import functools

import jax
import jax.numpy as jnp
from jax import lax
from jax.experimental import pallas as pl
from jax.experimental.pallas import tpu as pltpu
from jax.experimental.pallas import tpu_sc as plsc

VOCAB = 1000000
EMB = 64
NC = 2
NS = 16
NW = NC * NS
TOT = 16384 * 50
PER_W = TOT // NW
CHUNK = 128
NCHUNK = PER_W // CHUNK

_mesh = plsc.VectorSubcoreMesh(
    core_axis_name="c", subcore_axis_name="s", num_cores=NC, num_subcores=NS
)


@functools.partial(
    pl.kernel,
    out_type=jax.ShapeDtypeStruct((TOT, EMB), jnp.float32),
    mesh=_mesh,
    compiler_params=pltpu.CompilerParams(use_tc_tiling_on_sc=False),
    scratch_types=[
        pltpu.VMEM((NCHUNK, CHUNK), jnp.int32),
        pltpu.VMEM((2, CHUNK, EMB), jnp.float32),
        pltpu.SemaphoreType.DMA((2,)),
        pltpu.SemaphoreType.DMA((2,)),
    ],
)
def _embed_gather(idx_hbm, table_hbm, out_hbm, idx_v, rows_v, gsem, osem):
    wid = lax.axis_index("s") * NC + lax.axis_index("c")
    base = wid * PER_W
    pltpu.sync_copy(idx_hbm.at[wid], idx_v)

    def gather(j, slot):
        return pltpu.make_async_copy(
            table_hbm.at[idx_v.at[j]], rows_v.at[slot], gsem.at[slot]
        )

    def out_copy(j, slot):
        return pltpu.make_async_copy(
            rows_v.at[slot],
            out_hbm.at[pl.ds(base + j * CHUNK, CHUNK)],
            osem.at[slot],
        )

    gather(0, 0).start()

    @pl.loop(0, NCHUNK, step=2)
    def _(i):
        for b in range(2):
            j = i + b

            @pl.when(j >= 1)
            def _():
                out_copy(j - 1, 1 - b).wait()

            @pl.when(j + 1 < NCHUNK)
            def _():
                gather(j + 1, 1 - b).start()

            gather(j, b).wait()
            out_copy(j, b).start()

    out_copy(NCHUNK - 1, (NCHUNK - 1) % 2).wait()


def kernel(input, table):
    flat_idx = input.reshape(NW, NCHUNK, CHUNK)
    out = _embed_gather(flat_idx, table)
    return out.reshape(input.shape[0], input.shape[1], EMB)
